# baseline (speedup 1.0000x reference)
import jax
import jax.numpy as jnp
from jax import lax
from jax.experimental import pallas as pl
from jax.experimental.pallas import tpu as pltpu
from jax.experimental.pallas import tpu_sc as plsc

EMBED = 32
BATCH = 16384
NC = 2
NS = 16
NW = NC * NS
BPW = BATCH // NW
CHUNK = 128
NCHUNK = BPW // CHUNK
LANES = 16
NGROUP = BPW // LANES


def _tt_kernel(pos_h, pos_t, pos_r, pos_tem,
               neg_h, neg_t, neg_r, neg_tem,
               ent_w, rel_w, tem_w,
               pos_out, neg_out,
               idx_h, idx_t, idx_r, idx_tm,
               rows_h, rows_t, rows_r, rows_tm,
               out_v, sem):
    wid = lax.axis_index("s") * NC + lax.axis_index("c")
    base = wid * BPW
    iota = lax.iota(jnp.int32, LANES)

    iota32 = iota * EMBED

    def do_side(ih, it, ir, itm, out_hbm):
        pltpu.sync_copy(ih.at[pl.ds(base, BPW)], idx_h)
        pltpu.sync_copy(it.at[pl.ds(base, BPW)], idx_t)
        pltpu.sync_copy(ir.at[pl.ds(base, BPW)], idx_r)
        pltpu.sync_copy(itm.at[pl.ds(base, BPW)], idx_tm)
        cps = []
        for c in range(NCHUNK):
            sl = pl.ds(c * CHUNK, CHUNK)
            cps.append(pltpu.async_copy(ent_w.at[idx_h.at[sl]], rows_h.at[sl], sem))
            cps.append(pltpu.async_copy(ent_w.at[idx_t.at[sl]], rows_t.at[sl], sem))
            cps.append(pltpu.async_copy(rel_w.at[idx_r.at[sl]], rows_r.at[sl], sem))
            cps.append(pltpu.async_copy(tem_w.at[idx_tm.at[sl]], rows_tm.at[sl], sem))
        for cp in cps:
            cp.wait()

        def gbody(g, carry):
            row_idx = g * LANES + iota
            s = jnp.zeros((LANES,), jnp.float32)
            for j in range(EMBED):
                col = jnp.full((LANES,), j, jnp.int32)
                vh = plsc.load_gather(rows_h, [row_idx, col])
                vt = plsc.load_gather(rows_t, [row_idx, col])
                vr = plsc.load_gather(rows_r, [row_idx, col])
                vtm = plsc.load_gather(rows_tm, [row_idx, col])
                s = s + jnp.abs(vh + vr + vtm - vt)
            out_v[pl.ds(g * LANES, LANES)] = s
            return carry

        lax.fori_loop(0, NGROUP, gbody, 0)
        pltpu.sync_copy(out_v, out_hbm.at[pl.ds(base, BPW)])

    do_side(pos_h, pos_t, pos_r, pos_tem, pos_out)
    do_side(neg_h, neg_t, neg_r, neg_tem, neg_out)


def kernel(pos_h, pos_t, pos_r, pos_tem, neg_h, neg_t, neg_r, neg_tem,
           ent_w, rel_w, tem_w):
    mesh = plsc.VectorSubcoreMesh(core_axis_name="c", subcore_axis_name="s")
    f = pl.kernel(
        _tt_kernel,
        mesh=mesh,
        out_type=(
            jax.ShapeDtypeStruct((BATCH,), jnp.float32),
            jax.ShapeDtypeStruct((BATCH,), jnp.float32),
        ),
        scratch_types=[
            pltpu.VMEM((BPW,), jnp.int32),
            pltpu.VMEM((BPW,), jnp.int32),
            pltpu.VMEM((BPW,), jnp.int32),
            pltpu.VMEM((BPW,), jnp.int32),
            pltpu.VMEM((BPW, EMBED), jnp.float32),
            pltpu.VMEM((BPW, EMBED), jnp.float32),
            pltpu.VMEM((BPW, EMBED), jnp.float32),
            pltpu.VMEM((BPW, EMBED), jnp.float32),
            pltpu.VMEM((BPW,), jnp.float32),
            pltpu.SemaphoreType.DMA,
        ],
        compiler_params=pltpu.CompilerParams(
            needs_layout_passes=False, use_tc_tiling_on_sc=False),
    )
    i32 = jnp.int32
    return f(pos_h.astype(i32), pos_t.astype(i32), pos_r.astype(i32),
             pos_tem.astype(i32), neg_h.astype(i32), neg_t.astype(i32),
             neg_r.astype(i32), neg_tem.astype(i32),
             ent_w, rel_w, tem_w)
